# Initial kernel scaffold; baseline (speedup 1.0000x reference)
import jax
import jax.numpy as jnp
from jax.experimental import pallas as pl


def kernel(x, edge_index, edge_label_index, W1l, W1r, b1, g1, be1, m1, v1, W2l, W2r, b2, g2, be2, m2, v2, W3l, W3r, b3, Wd1, bd1, Wd2, bd2):
    raise NotImplementedError("write your pallas kernel here")


# speedup vs baseline: 4.1525x; 4.1525x over previous
import functools

import jax
import jax.numpy as jnp
from jax import lax
from jax.experimental import pallas as pl
from jax.experimental.pallas import tpu as pltpu
from jax.experimental.pallas import tpu_sc as plsc

N = 10000
E = 320000
EL = 200000
ELP = 204800
H = 128

NC = 2
NS = 16
NW = NC * NS
C = 80
ZR = 40
NRC = N // ZR

_MESH = plsc.VectorSubcoreMesh(core_axis_name="c", subcore_axis_name="s")


def _fill_vmem(ref, rows, width, value):
    per_row = width // 16

    def body(i, _):
        ref[i // per_row, pl.ds((i % per_row) * 16, 16)] = jnp.full(
            (16,), value, jnp.float32)
        return 0

    lax.fori_loop(0, rows * per_row, body, 0)


def _row_chunk_loop(s, fn):
    def body(j, _):
        fn(pl.multiple_of((j * NS + s) * ZR, 8))
        return 0

    lax.fori_loop(0, NRC // NS, body, 0)

    @pl.when(s < NRC % NS)
    def _():
        fn(pl.multiple_of(((NRC // NS) * NS + s) * ZR, 8))


def _sc_aggregate(x, src, dst):
    epw = E // NW
    nch = epw // C

    @functools.partial(
        pl.kernel,
        out_type=jax.ShapeDtypeStruct((NC, N, H), jnp.float32),
        mesh=_MESH,
        scratch_types=[
            pltpu.VMEM((C,), jnp.int32),
            pltpu.VMEM((C,), jnp.int32),
            pltpu.VMEM((C, H), jnp.float32),
            pltpu.VMEM((ZR, H), jnp.float32),
            pltpu.VMEM_SHARED((N, H), jnp.float32),
            pltpu.SemaphoreType.DMA,
        ])
    def k(x_hbm, src_hbm, dst_hbm, p_hbm, sidx, didx, buf, zbuf, acc_sh, sem):
        c = lax.axis_index("c")
        s = lax.axis_index("s")
        wid = s * NC + c

        _fill_vmem(zbuf, ZR, H, 0.0)
        _row_chunk_loop(
            s, lambda r: pltpu.sync_copy(zbuf, acc_sh.at[pl.ds(r, ZR)]))
        plsc.subcore_barrier()

        base0 = wid * epw

        def body(i, _):
            b = pl.multiple_of(base0 + i * C, 8)
            pltpu.sync_copy(src_hbm.at[pl.ds(b, C)], sidx)
            pltpu.sync_copy(dst_hbm.at[pl.ds(b, C)], didx)
            pltpu.async_copy(x_hbm.at[sidx], buf, sem).wait()
            pltpu.sync_copy(buf, acc_sh.at[didx], add=True)
            return 0

        lax.fori_loop(0, nch, body, 0)
        plsc.subcore_barrier()

        def dump(r):
            pltpu.sync_copy(acc_sh.at[pl.ds(r, ZR)], zbuf)
            pltpu.sync_copy(zbuf, p_hbm.at[c, pl.ds(r, ZR)])

        _row_chunk_loop(s, dump)

    return k(x, src, dst)


def _sc_degree(dst):
    epw = E // NW
    nch = epw // C

    @functools.partial(
        pl.kernel,
        out_type=jax.ShapeDtypeStruct((NC, N, H), jnp.float32),
        mesh=_MESH,
        scratch_types=[
            pltpu.VMEM((C,), jnp.int32),
            pltpu.VMEM((C, H), jnp.float32),
            pltpu.VMEM((ZR, H), jnp.float32),
            pltpu.VMEM_SHARED((N, H), jnp.float32),
        ])
    def k(dst_hbm, d_hbm, didx, ones, zbuf, acc_sh):
        c = lax.axis_index("c")
        s = lax.axis_index("s")
        wid = s * NC + c

        _fill_vmem(zbuf, ZR, H, 0.0)
        _row_chunk_loop(
            s, lambda r: pltpu.sync_copy(zbuf, acc_sh.at[pl.ds(r, ZR)]))
        _fill_vmem(ones, C, H, 1.0)
        plsc.subcore_barrier()

        base0 = wid * epw

        def body(i, _):
            b = pl.multiple_of(base0 + i * C, 8)
            pltpu.sync_copy(dst_hbm.at[pl.ds(b, C)], didx)
            pltpu.sync_copy(ones, acc_sh.at[didx], add=True)
            return 0

        lax.fori_loop(0, nch, body, 0)
        plsc.subcore_barrier()

        def dump(r):
            pltpu.sync_copy(acc_sh.at[pl.ds(r, ZR)], zbuf)
            pltpu.sync_copy(zbuf, d_hbm.at[c, pl.ds(r, ZR)])

        _row_chunk_loop(s, dump)

    return k(dst)


def _sc_decoder_gather(a, b, ls, ld):
    nchunks = EL // C
    full = nchunks // NW
    rem = nchunks - full * NW

    @functools.partial(
        pl.kernel,
        out_type=jax.ShapeDtypeStruct((ELP, H), jnp.float32),
        mesh=_MESH,
        scratch_types=[
            pltpu.VMEM((C,), jnp.int32),
            pltpu.VMEM((C,), jnp.int32),
            pltpu.VMEM((C, H), jnp.float32),
            pltpu.SemaphoreType.DMA,
        ])
    def k(a_hbm, b_hbm, ls_hbm, ld_hbm, s_hbm, lidx, didx, buf, sem):
        c = lax.axis_index("c")
        s = lax.axis_index("s")
        wid = s * NC + c

        def do_chunk(kk):
            off = pl.multiple_of(kk * C, 8)
            pltpu.sync_copy(ls_hbm.at[pl.ds(off, C)], lidx)
            pltpu.sync_copy(ld_hbm.at[pl.ds(off, C)], didx)
            pltpu.async_copy(a_hbm.at[lidx], buf, sem).wait()
            pltpu.async_copy(b_hbm.at[didx], buf, sem, add=True).wait()
            pltpu.sync_copy(buf, s_hbm.at[pl.ds(off, C)])

        def body(i, _):
            do_chunk(i * NW + wid)
            return 0

        lax.fori_loop(0, full, body, 0)

        @pl.when(wid < rem)
        def _():
            do_chunk(full * NW + wid)

    return k(a, b, ls, ld)


_ROWS = 1000


def _tc_layer1(P, Dp, x, Wl, Wr, bias, gamma, beta, mean, var):
    def body(p0, p1, d0, d1, xb, wl, wr, bv, gv, bev, mv, vv, ob, rd):
        deg = jnp.maximum(d0[0][:, 0:1] + d1[0][:, 0:1], 1.0)
        rdeg = 1.0 / deg
        agg = (p0[0] + p1[0]) * rdeg
        h = (lax.dot_general(agg, wl[...], (((1,), (1,)), ((), ())))
             + lax.dot_general(xb[...], wr[...], (((1,), (1,)), ((), ())))
             + bv[...])
        h = (h - mv[...]) * lax.rsqrt(vv[...] + 1e-5) * gv[...] + bev[...]
        ob[...] = jnp.maximum(h, 0.0)
        rd[...] = jnp.broadcast_to(rdeg, (rdeg.shape[0], 16))

    vec = pl.BlockSpec((1, H), lambda i: (0, 0))
    return pl.pallas_call(
        body,
        grid=(N // _ROWS,),
        in_specs=[
            pl.BlockSpec((1, _ROWS, H), lambda i: (0, i, 0)),
            pl.BlockSpec((1, _ROWS, H), lambda i: (1, i, 0)),
            pl.BlockSpec((1, _ROWS, H), lambda i: (0, i, 0)),
            pl.BlockSpec((1, _ROWS, H), lambda i: (1, i, 0)),
            pl.BlockSpec((_ROWS, H), lambda i: (i, 0)),
            pl.BlockSpec((H, H), lambda i: (0, 0)),
            pl.BlockSpec((H, H), lambda i: (0, 0)),
            vec, vec, vec, vec, vec,
        ],
        out_specs=[
            pl.BlockSpec((_ROWS, H), lambda i: (i, 0)),
            pl.BlockSpec((_ROWS, 16), lambda i: (i, 0)),
        ],
        out_shape=[
            jax.ShapeDtypeStruct((N, H), jnp.float32),
            jax.ShapeDtypeStruct((N, 16), jnp.float32),
        ],
    )(P, P, Dp, Dp, x, Wl, Wr, bias, gamma, beta, mean, var)


def _tc_layer2(P, rdeg, x, Wl, Wr, bias, gamma, beta, mean, var):
    def body(p0, p1, rd, xb, wl, wr, bv, gv, bev, mv, vv, ob):
        agg = (p0[0] + p1[0]) * rd[:, 0:1]
        h = (lax.dot_general(agg, wl[...], (((1,), (1,)), ((), ())))
             + lax.dot_general(xb[...], wr[...], (((1,), (1,)), ((), ())))
             + bv[...])
        h = (h - mv[...]) * lax.rsqrt(vv[...] + 1e-5) * gv[...] + bev[...]
        ob[...] = jnp.maximum(h, 0.0)

    vec = pl.BlockSpec((1, H), lambda i: (0, 0))
    return pl.pallas_call(
        body,
        grid=(N // _ROWS,),
        in_specs=[
            pl.BlockSpec((1, _ROWS, H), lambda i: (0, i, 0)),
            pl.BlockSpec((1, _ROWS, H), lambda i: (1, i, 0)),
            pl.BlockSpec((_ROWS, 16), lambda i: (i, 0)),
            pl.BlockSpec((_ROWS, H), lambda i: (i, 0)),
            pl.BlockSpec((H, H), lambda i: (0, 0)),
            pl.BlockSpec((H, H), lambda i: (0, 0)),
            vec, vec, vec, vec, vec,
        ],
        out_specs=pl.BlockSpec((_ROWS, H), lambda i: (i, 0)),
        out_shape=jax.ShapeDtypeStruct((N, H), jnp.float32),
    )(P, P, rdeg, x, Wl, Wr, bias, gamma, beta, mean, var)


def _tc_layer3_fused(P, rdeg, x, W3l, W3r, b3, Wd1a, Wd1b, bd1):
    def body(p0, p1, rd, xb, wl, wr, bv, wa, wb, bdv, a_ref, b_ref):
        agg = (p0[0] + p1[0]) * rd[:, 0:1]
        z = (lax.dot_general(agg, wl[...], (((1,), (1,)), ((), ())))
             + lax.dot_general(xb[...], wr[...], (((1,), (1,)), ((), ())))
             + bv[...])
        a_ref[...] = lax.dot_general(z, wa[...], (((1,), (1,)), ((), ()))) + bdv[...]
        b_ref[...] = lax.dot_general(z, wb[...], (((1,), (1,)), ((), ())))

    vec = pl.BlockSpec((1, H), lambda i: (0, 0))
    mat = pl.BlockSpec((H, H), lambda i: (0, 0))
    return pl.pallas_call(
        body,
        grid=(N // _ROWS,),
        in_specs=[
            pl.BlockSpec((1, _ROWS, H), lambda i: (0, i, 0)),
            pl.BlockSpec((1, _ROWS, H), lambda i: (1, i, 0)),
            pl.BlockSpec((_ROWS, 16), lambda i: (i, 0)),
            pl.BlockSpec((_ROWS, H), lambda i: (i, 0)),
            mat, mat, vec, mat, mat, vec,
        ],
        out_specs=[
            pl.BlockSpec((_ROWS, H), lambda i: (i, 0)),
            pl.BlockSpec((_ROWS, H), lambda i: (i, 0)),
        ],
        out_shape=[
            jax.ShapeDtypeStruct((N, H), jnp.float32),
            jax.ShapeDtypeStruct((N, H), jnp.float32),
        ],
    )(P, P, rdeg, x, W3l, W3r, b3, Wd1a, Wd1b, bd1)


_EROWS = 2048


def _tc_decoder_out(S, wd2, bd2):
    nb = ELP // _EROWS

    def body(sb, wv, b2, ob):
        h = jnp.maximum(sb[...], 0.0)
        ob[...] = jnp.sum(h * wv[...], axis=1) + b2[0, 0]

    out2 = pl.pallas_call(
        body,
        grid=(nb,),
        in_specs=[
            pl.BlockSpec((_EROWS, H), lambda i: (i, 0)),
            pl.BlockSpec((1, H), lambda i: (0, 0)),
            pl.BlockSpec((1, 1), lambda i: (0, 0)),
        ],
        out_specs=pl.BlockSpec((_EROWS,), lambda i: (i,)),
        out_shape=jax.ShapeDtypeStruct((ELP,), jnp.float32),
    )(S, wd2, bd2)
    return out2[:EL]


def kernel(x, edge_index, edge_label_index, W1l, W1r, b1, g1, be1, m1, v1,
           W2l, W2r, b2, g2, be2, m2, v2, W3l, W3r, b3, Wd1, bd1, Wd2, bd2):
    src, dst = edge_index[0], edge_index[1]
    ls, ld = edge_label_index[0], edge_label_index[1]
    row = lambda v: v.reshape(1, H)
    Wd1a, Wd1b = Wd1[:, :H], Wd1[:, H:]

    Dp = _sc_degree(dst)
    P1 = _sc_aggregate(x, src, dst)
    h1, rdeg = _tc_layer1(P1, Dp, x, W1l, W1r, row(b1), row(g1), row(be1),
                          row(m1), row(v1))
    P2 = _sc_aggregate(h1, src, dst)
    h2 = _tc_layer2(P2, rdeg, h1, W2l, W2r, row(b2), row(g2), row(be2),
                    row(m2), row(v2))
    P3 = _sc_aggregate(h2, src, dst)
    A, B = _tc_layer3_fused(P3, rdeg, h2, W3l, W3r, row(b3), Wd1a, Wd1b,
                            row(bd1))
    S = _sc_decoder_gather(A, B, ls, ld)
    return _tc_decoder_out(S, Wd2.reshape(1, H), bd2.reshape(1, 1))


# speedup vs baseline: 9.2593x; 2.2298x over previous
import functools

import jax
import jax.numpy as jnp
from jax import lax
from jax.experimental import pallas as pl
from jax.experimental.pallas import tpu as pltpu
from jax.experimental.pallas import tpu_sc as plsc

N = 10000
E = 320000
EL = 200000
ELP = 204800
H = 128

NC = 2
NS = 16
NW = NC * NS
C = 80
ZR = 40
NRC = N // ZR

_MESH = plsc.VectorSubcoreMesh(core_axis_name="c", subcore_axis_name="s")


def _fill_vmem(ref, rows, width, value):
    per_row = width // 16

    def body(i, _):
        ref[i // per_row, pl.ds((i % per_row) * 16, 16)] = jnp.full(
            (16,), value, jnp.float32)
        return 0

    lax.fori_loop(0, rows * per_row, body, 0)


def _row_chunk_loop(s, fn):
    def body(j, _):
        fn(pl.multiple_of((j * NS + s) * ZR, 8))
        return 0

    lax.fori_loop(0, NRC // NS, body, 0)

    @pl.when(s < NRC % NS)
    def _():
        fn(pl.multiple_of(((NRC // NS) * NS + s) * ZR, 8))


CA = 64
NCHT = E // CA
FULL = NCHT // NW
REM = NCHT - FULL * NW
NB = 4
NI = 8


def _sc_aggregate(x, src, dst):
    @functools.partial(
        pl.kernel,
        out_type=jax.ShapeDtypeStruct((NC, N, H), jnp.float32),
        mesh=_MESH,
        scratch_types=[
            pltpu.VMEM((NI, CA), jnp.int32),
            pltpu.VMEM((NI, CA), jnp.int32),
            pltpu.VMEM((NB, CA, H), jnp.float32),
            pltpu.VMEM((ZR, H), jnp.float32),
            pltpu.VMEM_SHARED((N, H), jnp.float32),
            pltpu.SemaphoreType.DMA((NB,)),
            pltpu.SemaphoreType.DMA((NB,)),
            pltpu.SemaphoreType.DMA((NB,)),
        ])
    def k(x_hbm, src_hbm, dst_hbm, p_hbm, sidx, didx, bufs, zbuf, acc_sh,
          sem_i, sem_g, sem_s):
        c = lax.axis_index("c")
        s = lax.axis_index("s")
        wid = s * NC + c

        _fill_vmem(zbuf, ZR, H, 0.0)
        _row_chunk_loop(
            s, lambda r: pltpu.sync_copy(zbuf, acc_sh.at[pl.ds(r, ZR)]))
        plsc.subcore_barrier()

        cb = wid * FULL

        def issue_idx(j):
            off = pl.multiple_of((cb + j) * CA, 8)
            pltpu.async_copy(src_hbm.at[pl.ds(off, CA)], sidx.at[j % NI],
                             sem_i.at[j % NB])
            pltpu.async_copy(dst_hbm.at[pl.ds(off, CA)], didx.at[j % NI],
                             sem_i.at[j % NB])

        def wait_idx(j):
            pltpu.make_async_copy(src_hbm.at[pl.ds(0, CA)], sidx.at[0],
                                  sem_i.at[j % NB]).wait()
            pltpu.make_async_copy(dst_hbm.at[pl.ds(0, CA)], didx.at[0],
                                  sem_i.at[j % NB]).wait()

        def start_gather(j):
            pltpu.async_copy(x_hbm.at[sidx.at[j % NI]], bufs.at[j % NB],
                             sem_g.at[j % NB])

        def wait_gather(j):
            pltpu.make_async_copy(x_hbm.at[sidx.at[j % NI]], bufs.at[j % NB],
                                  sem_g.at[j % NB]).wait()

        def start_scatter(j):
            pltpu.async_copy(bufs.at[j % NB], acc_sh.at[didx.at[j % NI]],
                             sem_s.at[j % NB], add=True)

        def wait_scatter(j):
            pltpu.make_async_copy(bufs.at[j % NB],
                                  acc_sh.at[didx.at[j % NI]],
                                  sem_s.at[j % NB]).wait()

        issue_idx(0)
        issue_idx(1)

        def body(i, _):
            @pl.when(i + 2 < FULL)
            def _():
                issue_idx(i + 2)

            @pl.when(i >= NB)
            def _():
                wait_scatter(i - NB)

            wait_idx(i)
            start_gather(i)

            @pl.when(i >= 2)
            def _():
                wait_gather(i - 2)
                start_scatter(i - 2)

            return 0

        lax.fori_loop(0, FULL, body, 0)
        for j in (FULL - 2, FULL - 1):
            wait_gather(j)
            start_scatter(j)
        for j in range(FULL - NB, FULL):
            wait_scatter(j)

        @pl.when(wid < REM)
        def _():
            off = pl.multiple_of((NW * FULL + wid) * CA, 8)
            pltpu.sync_copy(src_hbm.at[pl.ds(off, CA)], sidx.at[0])
            pltpu.sync_copy(dst_hbm.at[pl.ds(off, CA)], didx.at[0])
            pltpu.async_copy(x_hbm.at[sidx.at[0]], bufs.at[0],
                             sem_g.at[0]).wait()
            pltpu.sync_copy(bufs.at[0], acc_sh.at[didx.at[0]], add=True)

        plsc.subcore_barrier()

        def dump(r):
            pltpu.sync_copy(acc_sh.at[pl.ds(r, ZR)], zbuf)
            pltpu.sync_copy(zbuf, p_hbm.at[c, pl.ds(r, ZR)])

        _row_chunk_loop(s, dump)

    return k(x, src, dst)


def _sc_degree(dst):
    @functools.partial(
        pl.kernel,
        out_type=jax.ShapeDtypeStruct((NC, N, H), jnp.float32),
        mesh=_MESH,
        scratch_types=[
            pltpu.VMEM((NI, CA), jnp.int32),
            pltpu.VMEM((CA, H), jnp.float32),
            pltpu.VMEM((ZR, H), jnp.float32),
            pltpu.VMEM_SHARED((N, H), jnp.float32),
            pltpu.SemaphoreType.DMA((NB,)),
            pltpu.SemaphoreType.DMA((NB,)),
        ])
    def k(dst_hbm, d_hbm, didx, ones, zbuf, acc_sh, sem_i, sem_s):
        c = lax.axis_index("c")
        s = lax.axis_index("s")
        wid = s * NC + c

        _fill_vmem(zbuf, ZR, H, 0.0)
        _row_chunk_loop(
            s, lambda r: pltpu.sync_copy(zbuf, acc_sh.at[pl.ds(r, ZR)]))
        _fill_vmem(ones, CA, H, 1.0)
        plsc.subcore_barrier()

        cb = wid * FULL

        def issue_idx(j):
            off = pl.multiple_of((cb + j) * CA, 8)
            pltpu.async_copy(dst_hbm.at[pl.ds(off, CA)], didx.at[j % NI],
                             sem_i.at[j % NB])

        def wait_idx(j):
            pltpu.make_async_copy(dst_hbm.at[pl.ds(0, CA)], didx.at[0],
                                  sem_i.at[j % NB]).wait()

        def start_scatter(j):
            pltpu.async_copy(ones, acc_sh.at[didx.at[j % NI]],
                             sem_s.at[j % NB], add=True)

        def wait_scatter(j):
            pltpu.make_async_copy(ones, acc_sh.at[didx.at[j % NI]],
                                  sem_s.at[j % NB]).wait()

        issue_idx(0)
        issue_idx(1)

        def body(i, _):
            @pl.when(i + 2 < FULL)
            def _():
                issue_idx(i + 2)

            @pl.when(i >= NB)
            def _():
                wait_scatter(i - NB)

            wait_idx(i)
            start_scatter(i)
            return 0

        lax.fori_loop(0, FULL, body, 0)
        for j in range(FULL - NB, FULL):
            wait_scatter(j)

        @pl.when(wid < REM)
        def _():
            off = pl.multiple_of((NW * FULL + wid) * CA, 8)
            pltpu.sync_copy(dst_hbm.at[pl.ds(off, CA)], didx.at[0])
            pltpu.sync_copy(ones, acc_sh.at[didx.at[0]], add=True)

        plsc.subcore_barrier()

        def dump(r):
            pltpu.sync_copy(acc_sh.at[pl.ds(r, ZR)], zbuf)
            pltpu.sync_copy(zbuf, d_hbm.at[c, pl.ds(r, ZR)])

        _row_chunk_loop(s, dump)

    return k(dst)


def _sc_decoder_gather(a, b, ls, ld):
    nchunks = EL // C
    full = nchunks // NW
    rem = nchunks - full * NW

    @functools.partial(
        pl.kernel,
        out_type=jax.ShapeDtypeStruct((ELP, H), jnp.float32),
        mesh=_MESH,
        scratch_types=[
            pltpu.VMEM((NI, C), jnp.int32),
            pltpu.VMEM((NI, C), jnp.int32),
            pltpu.VMEM((NB, C, H), jnp.float32),
            pltpu.SemaphoreType.DMA((NB,)),
            pltpu.SemaphoreType.DMA((NB,)),
            pltpu.SemaphoreType.DMA((NB,)),
            pltpu.SemaphoreType.DMA((NB,)),
        ])
    def k(a_hbm, b_hbm, ls_hbm, ld_hbm, s_hbm, lidx, didx, bufs,
          sem_i, sem_a, sem_b, sem_o):
        c = lax.axis_index("c")
        s = lax.axis_index("s")
        wid = s * NC + c

        def chunk_off(j):
            return pl.multiple_of((j * NW + wid) * C, 8)

        def issue_idx(j):
            off = chunk_off(j)
            pltpu.async_copy(ls_hbm.at[pl.ds(off, C)], lidx.at[j % NI],
                             sem_i.at[j % NB])
            pltpu.async_copy(ld_hbm.at[pl.ds(off, C)], didx.at[j % NI],
                             sem_i.at[j % NB])

        def wait_idx(j):
            pltpu.make_async_copy(ls_hbm.at[pl.ds(0, C)], lidx.at[0],
                                  sem_i.at[j % NB]).wait()
            pltpu.make_async_copy(ld_hbm.at[pl.ds(0, C)], didx.at[0],
                                  sem_i.at[j % NB]).wait()

        def start_a(j):
            pltpu.async_copy(a_hbm.at[lidx.at[j % NI]], bufs.at[j % NB],
                             sem_a.at[j % NB])

        def wait_a(j):
            pltpu.make_async_copy(a_hbm.at[lidx.at[j % NI]], bufs.at[j % NB],
                                  sem_a.at[j % NB]).wait()

        def start_b(j):
            pltpu.async_copy(b_hbm.at[didx.at[j % NI]], bufs.at[j % NB],
                             sem_b.at[j % NB], add=True)

        def wait_b(j):
            pltpu.make_async_copy(b_hbm.at[didx.at[j % NI]], bufs.at[j % NB],
                                  sem_b.at[j % NB]).wait()

        def start_store(j):
            pltpu.async_copy(bufs.at[j % NB], s_hbm.at[pl.ds(chunk_off(j), C)],
                             sem_o.at[j % NB])

        def wait_store(j):
            pltpu.make_async_copy(bufs.at[j % NB],
                                  s_hbm.at[pl.ds(chunk_off(j), C)],
                                  sem_o.at[j % NB]).wait()

        issue_idx(0)
        issue_idx(1)

        def body(i, _):
            @pl.when(i + 2 < full)
            def _():
                issue_idx(i + 2)

            @pl.when(i >= NB)
            def _():
                wait_store(i - NB)

            wait_idx(i)
            start_a(i)

            @pl.when(i >= 1)
            def _():
                wait_a(i - 1)
                start_b(i - 1)

            @pl.when(i >= 2)
            def _():
                wait_b(i - 2)
                start_store(i - 2)

            return 0

        lax.fori_loop(0, full, body, 0)
        wait_a(full - 1)
        start_b(full - 1)
        for j in (full - 2, full - 1):
            wait_b(j)
            start_store(j)
        for j in range(full - NB, full):
            wait_store(j)

        @pl.when(wid < rem)
        def _():
            off = pl.multiple_of((full * NW + wid) * C, 8)
            pltpu.sync_copy(ls_hbm.at[pl.ds(off, C)], lidx.at[0])
            pltpu.sync_copy(ld_hbm.at[pl.ds(off, C)], didx.at[0])
            pltpu.async_copy(a_hbm.at[lidx.at[0]], bufs.at[0],
                             sem_a.at[0]).wait()
            pltpu.async_copy(b_hbm.at[didx.at[0]], bufs.at[0],
                             sem_b.at[0], add=True).wait()
            pltpu.sync_copy(bufs.at[0], s_hbm.at[pl.ds(off, C)])

    return k(a, b, ls, ld)


_ROWS = 1000


def _tc_layer1(P, Dp, x, Wl, Wr, bias, gamma, beta, mean, var):
    def body(p0, p1, d0, d1, xb, wl, wr, bv, gv, bev, mv, vv, ob, rd):
        deg = jnp.maximum(d0[0][:, 0:1] + d1[0][:, 0:1], 1.0)
        rdeg = 1.0 / deg
        agg = (p0[0] + p1[0]) * rdeg
        h = (lax.dot_general(agg, wl[...], (((1,), (1,)), ((), ())))
             + lax.dot_general(xb[...], wr[...], (((1,), (1,)), ((), ())))
             + bv[...])
        h = (h - mv[...]) * lax.rsqrt(vv[...] + 1e-5) * gv[...] + bev[...]
        ob[...] = jnp.maximum(h, 0.0)
        rd[...] = jnp.broadcast_to(rdeg, (rdeg.shape[0], 16))

    vec = pl.BlockSpec((1, H), lambda i: (0, 0))
    return pl.pallas_call(
        body,
        grid=(N // _ROWS,),
        in_specs=[
            pl.BlockSpec((1, _ROWS, H), lambda i: (0, i, 0)),
            pl.BlockSpec((1, _ROWS, H), lambda i: (1, i, 0)),
            pl.BlockSpec((1, _ROWS, H), lambda i: (0, i, 0)),
            pl.BlockSpec((1, _ROWS, H), lambda i: (1, i, 0)),
            pl.BlockSpec((_ROWS, H), lambda i: (i, 0)),
            pl.BlockSpec((H, H), lambda i: (0, 0)),
            pl.BlockSpec((H, H), lambda i: (0, 0)),
            vec, vec, vec, vec, vec,
        ],
        out_specs=[
            pl.BlockSpec((_ROWS, H), lambda i: (i, 0)),
            pl.BlockSpec((_ROWS, 16), lambda i: (i, 0)),
        ],
        out_shape=[
            jax.ShapeDtypeStruct((N, H), jnp.float32),
            jax.ShapeDtypeStruct((N, 16), jnp.float32),
        ],
    )(P, P, Dp, Dp, x, Wl, Wr, bias, gamma, beta, mean, var)


def _tc_layer2(P, rdeg, x, Wl, Wr, bias, gamma, beta, mean, var):
    def body(p0, p1, rd, xb, wl, wr, bv, gv, bev, mv, vv, ob):
        agg = (p0[0] + p1[0]) * rd[:, 0:1]
        h = (lax.dot_general(agg, wl[...], (((1,), (1,)), ((), ())))
             + lax.dot_general(xb[...], wr[...], (((1,), (1,)), ((), ())))
             + bv[...])
        h = (h - mv[...]) * lax.rsqrt(vv[...] + 1e-5) * gv[...] + bev[...]
        ob[...] = jnp.maximum(h, 0.0)

    vec = pl.BlockSpec((1, H), lambda i: (0, 0))
    return pl.pallas_call(
        body,
        grid=(N // _ROWS,),
        in_specs=[
            pl.BlockSpec((1, _ROWS, H), lambda i: (0, i, 0)),
            pl.BlockSpec((1, _ROWS, H), lambda i: (1, i, 0)),
            pl.BlockSpec((_ROWS, 16), lambda i: (i, 0)),
            pl.BlockSpec((_ROWS, H), lambda i: (i, 0)),
            pl.BlockSpec((H, H), lambda i: (0, 0)),
            pl.BlockSpec((H, H), lambda i: (0, 0)),
            vec, vec, vec, vec, vec,
        ],
        out_specs=pl.BlockSpec((_ROWS, H), lambda i: (i, 0)),
        out_shape=jax.ShapeDtypeStruct((N, H), jnp.float32),
    )(P, P, rdeg, x, Wl, Wr, bias, gamma, beta, mean, var)


def _tc_layer3_fused(P, rdeg, x, W3l, W3r, b3, Wd1a, Wd1b, bd1):
    def body(p0, p1, rd, xb, wl, wr, bv, wa, wb, bdv, a_ref, b_ref):
        agg = (p0[0] + p1[0]) * rd[:, 0:1]
        z = (lax.dot_general(agg, wl[...], (((1,), (1,)), ((), ())))
             + lax.dot_general(xb[...], wr[...], (((1,), (1,)), ((), ())))
             + bv[...])
        a_ref[...] = lax.dot_general(z, wa[...], (((1,), (1,)), ((), ()))) + bdv[...]
        b_ref[...] = lax.dot_general(z, wb[...], (((1,), (1,)), ((), ())))

    vec = pl.BlockSpec((1, H), lambda i: (0, 0))
    mat = pl.BlockSpec((H, H), lambda i: (0, 0))
    return pl.pallas_call(
        body,
        grid=(N // _ROWS,),
        in_specs=[
            pl.BlockSpec((1, _ROWS, H), lambda i: (0, i, 0)),
            pl.BlockSpec((1, _ROWS, H), lambda i: (1, i, 0)),
            pl.BlockSpec((_ROWS, 16), lambda i: (i, 0)),
            pl.BlockSpec((_ROWS, H), lambda i: (i, 0)),
            mat, mat, vec, mat, mat, vec,
        ],
        out_specs=[
            pl.BlockSpec((_ROWS, H), lambda i: (i, 0)),
            pl.BlockSpec((_ROWS, H), lambda i: (i, 0)),
        ],
        out_shape=[
            jax.ShapeDtypeStruct((N, H), jnp.float32),
            jax.ShapeDtypeStruct((N, H), jnp.float32),
        ],
    )(P, P, rdeg, x, W3l, W3r, b3, Wd1a, Wd1b, bd1)


_EROWS = 2048


def _tc_decoder_out(S, wd2, bd2):
    nb = ELP // _EROWS

    def body(sb, wv, b2, ob):
        h = jnp.maximum(sb[...], 0.0)
        ob[...] = jnp.sum(h * wv[...], axis=1) + b2[0, 0]

    out2 = pl.pallas_call(
        body,
        grid=(nb,),
        in_specs=[
            pl.BlockSpec((_EROWS, H), lambda i: (i, 0)),
            pl.BlockSpec((1, H), lambda i: (0, 0)),
            pl.BlockSpec((1, 1), lambda i: (0, 0)),
        ],
        out_specs=pl.BlockSpec((_EROWS,), lambda i: (i,)),
        out_shape=jax.ShapeDtypeStruct((ELP,), jnp.float32),
    )(S, wd2, bd2)
    return out2[:EL]


def kernel(x, edge_index, edge_label_index, W1l, W1r, b1, g1, be1, m1, v1,
           W2l, W2r, b2, g2, be2, m2, v2, W3l, W3r, b3, Wd1, bd1, Wd2, bd2):
    src, dst = edge_index[0], edge_index[1]
    ls, ld = edge_label_index[0], edge_label_index[1]
    row = lambda v: v.reshape(1, H)
    Wd1a, Wd1b = Wd1[:, :H], Wd1[:, H:]

    Dp = _sc_degree(dst)
    P1 = _sc_aggregate(x, src, dst)
    h1, rdeg = _tc_layer1(P1, Dp, x, W1l, W1r, row(b1), row(g1), row(be1),
                          row(m1), row(v1))
    P2 = _sc_aggregate(h1, src, dst)
    h2 = _tc_layer2(P2, rdeg, h1, W2l, W2r, row(b2), row(g2), row(be2),
                    row(m2), row(v2))
    P3 = _sc_aggregate(h2, src, dst)
    A, B = _tc_layer3_fused(P3, rdeg, h2, W3l, W3r, row(b3), Wd1a, Wd1b,
                            row(bd1))
    S = _sc_decoder_gather(A, B, ls, ld)
    return _tc_decoder_out(S, Wd2.reshape(1, H), bd2.reshape(1, 1))
